# Initial kernel scaffold; baseline (speedup 1.0000x reference)
import jax
import jax.numpy as jnp
from jax.experimental import pallas as pl


def kernel(x, edge_index, W1, b1, W2, b2, W3, b3, W4, b4):
    raise NotImplementedError("write your pallas kernel here")


# speedup vs baseline: 11.1173x; 11.1173x over previous
import jax
import jax.numpy as jnp
from jax import lax
from jax.experimental import pallas as pl
from jax.experimental.pallas import tpu as pltpu
from jax.experimental.pallas import tpu_sc as plsc

N = 10000
E = 320000
D = 128

NC = 2
NS = 16
NW = NC * NS
EPW = E // NW
K = 80
NCHUNK = EPW // K
RSTRIPE = 624
RTAIL = N - NS * RSTRIPE


def _stripe_copy(src_at, dst_at, sid):
    pltpu.sync_copy(src_at(sid * RSTRIPE, RSTRIPE), dst_at(sid * RSTRIPE, RSTRIPE))

    @pl.when(sid == NS - 1)
    def _():
        pltpu.sync_copy(src_at(NS * RSTRIPE, RTAIL), dst_at(NS * RSTRIPE, RTAIL))


def _sc_scatter_body(g_hbm, src4d_hbm, dst4d_hbm, zeros_hbm, out_hbm,
                     src_slab, dst_idx, rows_v, acc_sh):
    cid = lax.axis_index("c")
    sid = lax.axis_index("s")
    wid = sid * NC + cid

    _stripe_copy(lambda o, n: zeros_hbm.at[pl.ds(o, n)],
                 lambda o, n: acc_sh.at[pl.ds(o, n)], sid)

    pltpu.sync_copy(src4d_hbm.at[wid], src_slab)

    plsc.subcore_barrier()

    def chunk(c, _):
        pltpu.sync_copy(dst4d_hbm.at[wid, c, 0], dst_idx)
        pltpu.sync_copy(g_hbm.at[src_slab.at[c, 0]], rows_v)
        pltpu.sync_copy(rows_v, acc_sh.at[dst_idx], add=True)
        return _

    lax.fori_loop(0, NCHUNK, chunk, None)

    plsc.subcore_barrier()

    _stripe_copy(lambda o, n: acc_sh.at[pl.ds(o, n)],
                 lambda o, n: out_hbm.at[cid, pl.ds(o, n)], sid)


_sc_scatter = pl.kernel(
    _sc_scatter_body,
    out_type=jax.ShapeDtypeStruct((NC, N, D), jnp.float32),
    mesh=plsc.VectorSubcoreMesh(core_axis_name="c", subcore_axis_name="s"),
    scratch_types=[
        pltpu.VMEM((NCHUNK, 1, K), jnp.int32),
        pltpu.VMEM((K,), jnp.int32),
        pltpu.VMEM((K, D), jnp.float32),
        pltpu.VMEM_SHARED((N, D), jnp.float32),
    ],
)


def _sc_degree_body(dst4d_hbm, zeros_hbm, ones_hbm, out_hbm,
                    dst_idx, ones_v, acc_sh):
    cid = lax.axis_index("c")
    sid = lax.axis_index("s")
    wid = sid * NC + cid

    pltpu.sync_copy(ones_hbm, ones_v)

    _stripe_copy(lambda o, n: zeros_hbm.at[pl.ds(o, n)],
                 lambda o, n: acc_sh.at[pl.ds(o, n)], sid)

    plsc.subcore_barrier()

    def chunk(c, _):
        pltpu.sync_copy(dst4d_hbm.at[wid, c, 0], dst_idx)
        pltpu.sync_copy(ones_v, acc_sh.at[dst_idx], add=True)
        return _

    lax.fori_loop(0, NCHUNK, chunk, None)

    plsc.subcore_barrier()

    _stripe_copy(lambda o, n: acc_sh.at[pl.ds(o, n)],
                 lambda o, n: out_hbm.at[cid, pl.ds(o, n)], sid)


_sc_degree = pl.kernel(
    _sc_degree_body,
    out_type=jax.ShapeDtypeStruct((NC, N, D), jnp.float32),
    mesh=plsc.VectorSubcoreMesh(core_axis_name="c", subcore_axis_name="s"),
    scratch_types=[
        pltpu.VMEM((K,), jnp.int32),
        pltpu.VMEM((K, D), jnp.float32),
        pltpu.VMEM_SHARED((N, D), jnp.float32),
    ],
)


def _dinv(degp_ref):
    deg = degp_ref[0, :, 0:1] + degp_ref[1, :, 0:1] + 1.0
    return lax.rsqrt(deg)


def _tc_first_body(degp_ref, x_ref, w_ref, out_ref):
    dinv = _dinv(degp_ref)
    out_ref[...] = jnp.dot(dinv * x_ref[...], w_ref[...],
                           preferred_element_type=jnp.float32)


def _tc_mid_body(degp_ref, g_ref, sp_ref, b_ref, w_ref, out_ref):
    dinv = _dinv(degp_ref)
    h = dinv * (g_ref[...] + sp_ref[0] + sp_ref[1]) + b_ref[...]
    xn = jnp.where(h > 0, h, 0.1 * h)
    out_ref[...] = jnp.dot(dinv * xn, w_ref[...],
                           preferred_element_type=jnp.float32)


def _tc_final_body(degp_ref, g_ref, sp_ref, b_ref, out_ref):
    dinv = _dinv(degp_ref)
    h = dinv * (g_ref[...] + sp_ref[0] + sp_ref[1]) + b_ref[...]
    out_ref[...] = jnp.sum(h, axis=1, keepdims=True) * (1.0 / D)


_tc_first = pl.pallas_call(
    _tc_first_body, out_shape=jax.ShapeDtypeStruct((N, D), jnp.float32))
_tc_mid = pl.pallas_call(
    _tc_mid_body, out_shape=jax.ShapeDtypeStruct((N, D), jnp.float32))
_tc_final = pl.pallas_call(
    _tc_final_body, out_shape=jax.ShapeDtypeStruct((N, 1), jnp.float32))


def kernel(x, edge_index, W1, b1, W2, b2, W3, b3, W4, b4):
    src4d = edge_index[0].reshape(NW, NCHUNK, 1, K)
    dst4d = edge_index[1].reshape(NW, NCHUNK, 1, K)
    zeros = jnp.zeros((N, D), jnp.float32)
    ones = jnp.ones((K, D), jnp.float32)

    degp = _sc_degree(dst4d, zeros, ones)

    g = _tc_first(degp, x, W1)
    for b, w in ((b1, W2), (b2, W3), (b3, W4)):
        sp = _sc_scatter(g, src4d, dst4d, zeros)
        g = _tc_mid(degp, g, sp, b.reshape(1, D), w)
    sp = _sc_scatter(g, src4d, dst4d, zeros)
    out = _tc_final(degp, g, sp, b4.reshape(1, D))
    return out.reshape(N)


# speedup vs baseline: 12.3727x; 1.1129x over previous
import jax
import jax.numpy as jnp
from jax import lax
from jax.experimental import pallas as pl
from jax.experimental.pallas import tpu as pltpu
from jax.experimental.pallas import tpu_sc as plsc

N = 10000
E = 320000
D = 128

NC = 2
NS = 16
NW = NC * NS
EPW = E // NW
K = 80
NCHUNK = EPW // K
RSTRIPE = 624
RTAIL = N - NS * RSTRIPE


def _stripe_copy(src_at, dst_at, sid):
    pltpu.sync_copy(src_at(sid * RSTRIPE, RSTRIPE), dst_at(sid * RSTRIPE, RSTRIPE))

    @pl.when(sid == NS - 1)
    def _():
        pltpu.sync_copy(src_at(NS * RSTRIPE, RTAIL), dst_at(NS * RSTRIPE, RTAIL))


NB = 5
NROUND = NCHUNK // NB


def _sc_scatter_body(g_hbm, src4d_hbm, dst4d_hbm, zeros_hbm, out_hbm,
                     src_slab, dst_slab, rows_v, acc_sh):
    cid = lax.axis_index("c")
    sid = lax.axis_index("s")
    wid = sid * NC + cid

    pltpu.sync_copy(src4d_hbm.at[wid], src_slab)
    pltpu.sync_copy(dst4d_hbm.at[wid], dst_slab)

    _stripe_copy(lambda o, n: zeros_hbm.at[pl.ds(o, n)],
                 lambda o, n: acc_sh.at[pl.ds(o, n)], sid)

    plsc.subcore_barrier()

    def chunk(c, _):
        pltpu.sync_copy(g_hbm.at[src_slab.at[c, 0]], rows_v)
        pltpu.sync_copy(rows_v, acc_sh.at[dst_slab.at[c, 0]], add=True)
        return _

    lax.fori_loop(0, NCHUNK, chunk, None)

    plsc.subcore_barrier()

    _stripe_copy(lambda o, n: acc_sh.at[pl.ds(o, n)],
                 lambda o, n: out_hbm.at[cid, pl.ds(o, n)], sid)


_sc_scatter = pl.kernel(
    _sc_scatter_body,
    out_type=jax.ShapeDtypeStruct((NC, N, D), jnp.float32),
    mesh=plsc.VectorSubcoreMesh(core_axis_name="c", subcore_axis_name="s"),
    name="sc_edge_scatter",
    scratch_types=[
        pltpu.VMEM((NCHUNK, 1, K), jnp.int32),
        pltpu.VMEM((NCHUNK, 1, K), jnp.int32),
        pltpu.VMEM((K, D), jnp.float32),
        pltpu.VMEM_SHARED((N, D), jnp.float32),
    ],
)


def _dinv(degp_ref):
    deg = degp_ref[0, :, 0:1] + degp_ref[1, :, 0:1] + 1.0
    return lax.rsqrt(deg)


def _tc_first_body(degp_ref, x_ref, w_ref, out_ref):
    dinv = _dinv(degp_ref)
    out_ref[...] = jnp.dot(dinv * x_ref[...], w_ref[...],
                           preferred_element_type=jnp.float32)


def _tc_mid_body(degp_ref, g_ref, sp_ref, b_ref, w_ref, out_ref):
    dinv = _dinv(degp_ref)
    h = dinv * (g_ref[...] + sp_ref[0] + sp_ref[1]) + b_ref[...]
    xn = jnp.where(h > 0, h, 0.1 * h)
    out_ref[...] = jnp.dot(dinv * xn, w_ref[...],
                           preferred_element_type=jnp.float32)


def _tc_final_body(degp_ref, g_ref, sp_ref, b_ref, out_ref):
    dinv = _dinv(degp_ref)
    h = dinv * (g_ref[...] + sp_ref[0] + sp_ref[1]) + b_ref[...]
    out_ref[...] = jnp.sum(h, axis=1, keepdims=True) * (1.0 / D)


_tc_first = pl.pallas_call(
    _tc_first_body, out_shape=jax.ShapeDtypeStruct((N, D), jnp.float32))
_tc_mid = pl.pallas_call(
    _tc_mid_body, out_shape=jax.ShapeDtypeStruct((N, D), jnp.float32))
_tc_final = pl.pallas_call(
    _tc_final_body, out_shape=jax.ShapeDtypeStruct((N, 1), jnp.float32))


def kernel(x, edge_index, W1, b1, W2, b2, W3, b3, W4, b4):
    src4d = edge_index[0].reshape(NW, NCHUNK, 1, K)
    dst4d = edge_index[1].reshape(NW, NCHUNK, 1, K)
    zeros = x * 0.0
    ones_nd = zeros + 1.0

    degp = _sc_scatter(ones_nd, src4d, dst4d, zeros)

    g = _tc_first(degp, x, W1)
    for b, w in ((b1, W2), (b2, W3), (b3, W4)):
        sp = _sc_scatter(g, src4d, dst4d, zeros)
        g = _tc_mid(degp, g, sp, b.reshape(1, D), w)
    sp = _sc_scatter(g, src4d, dst4d, zeros)
    out = _tc_final(degp, g, sp, b4.reshape(1, D))
    return out.reshape(N)


# speedup vs baseline: 12.3798x; 1.0006x over previous
import jax
import jax.numpy as jnp
from jax import lax
from jax.experimental import pallas as pl
from jax.experimental.pallas import tpu as pltpu
from jax.experimental.pallas import tpu_sc as plsc

N = 10000
E = 320000
D = 128

NC = 2
NS = 16
NW = NC * NS
EPW = E // NW
K = 80
NCHUNK = EPW // K
RSTRIPE = 624
RTAIL = N - NS * RSTRIPE


def _stripe_copy(src_at, dst_at, sid):
    pltpu.sync_copy(src_at(sid * RSTRIPE, RSTRIPE), dst_at(sid * RSTRIPE, RSTRIPE))

    @pl.when(sid == NS - 1)
    def _():
        pltpu.sync_copy(src_at(NS * RSTRIPE, RTAIL), dst_at(NS * RSTRIPE, RTAIL))


NB = 5
NROUND = NCHUNK // NB


def _sc_scatter_body(g_hbm, src4d_hbm, dst4d_hbm, zeros_hbm, out_hbm,
                     src_slab, dst_slab, acc_sh):
    cid = lax.axis_index("c")
    sid = lax.axis_index("s")
    wid = sid * NC + cid

    pltpu.sync_copy(src4d_hbm.at[wid], src_slab)
    pltpu.sync_copy(dst4d_hbm.at[wid], dst_slab)

    _stripe_copy(lambda o, n: zeros_hbm.at[pl.ds(o, n)],
                 lambda o, n: acc_sh.at[pl.ds(o, n)], sid)

    plsc.subcore_barrier()

    @plsc.parallel_loop(0, NCHUNK, unroll=5)
    def chunk(c):
        def inner(rows_v):
            pltpu.sync_copy(g_hbm.at[src_slab.at[c, 0]], rows_v)
            pltpu.sync_copy(rows_v, acc_sh.at[dst_slab.at[c, 0]], add=True)

        pl.run_scoped(inner, pltpu.VMEM((K, D), jnp.float32))

    plsc.subcore_barrier()

    _stripe_copy(lambda o, n: acc_sh.at[pl.ds(o, n)],
                 lambda o, n: out_hbm.at[cid, pl.ds(o, n)], sid)


_sc_scatter = pl.kernel(
    _sc_scatter_body,
    out_type=jax.ShapeDtypeStruct((NC, N, D), jnp.float32),
    mesh=plsc.VectorSubcoreMesh(core_axis_name="c", subcore_axis_name="s"),
    name="sc_edge_scatter",
    scratch_types=[
        pltpu.VMEM((NCHUNK, 1, K), jnp.int32),
        pltpu.VMEM((NCHUNK, 1, K), jnp.int32),
        pltpu.VMEM_SHARED((N, D), jnp.float32),
    ],
)


def _dinv(degp_ref):
    deg = degp_ref[0, :, 0:1] + degp_ref[1, :, 0:1] + 1.0
    return lax.rsqrt(deg)


def _tc_first_body(degp_ref, x_ref, w_ref, out_ref):
    dinv = _dinv(degp_ref)
    out_ref[...] = jnp.dot(dinv * x_ref[...], w_ref[...],
                           preferred_element_type=jnp.float32)


def _tc_mid_body(degp_ref, g_ref, sp_ref, b_ref, w_ref, out_ref):
    dinv = _dinv(degp_ref)
    h = dinv * (g_ref[...] + sp_ref[0] + sp_ref[1]) + b_ref[...]
    xn = jnp.where(h > 0, h, 0.1 * h)
    out_ref[...] = jnp.dot(dinv * xn, w_ref[...],
                           preferred_element_type=jnp.float32)


def _tc_final_body(degp_ref, g_ref, sp_ref, b_ref, out_ref):
    dinv = _dinv(degp_ref)
    h = dinv * (g_ref[...] + sp_ref[0] + sp_ref[1]) + b_ref[...]
    out_ref[...] = jnp.sum(h, axis=1, keepdims=True) * (1.0 / D)


_tc_first = pl.pallas_call(
    _tc_first_body, out_shape=jax.ShapeDtypeStruct((N, D), jnp.float32))
_tc_mid = pl.pallas_call(
    _tc_mid_body, out_shape=jax.ShapeDtypeStruct((N, D), jnp.float32))
_tc_final = pl.pallas_call(
    _tc_final_body, out_shape=jax.ShapeDtypeStruct((N, 1), jnp.float32))


def kernel(x, edge_index, W1, b1, W2, b2, W3, b3, W4, b4):
    src4d = edge_index[0].reshape(NW, NCHUNK, 1, K)
    dst4d = edge_index[1].reshape(NW, NCHUNK, 1, K)
    zeros = x * 0.0
    ones_nd = zeros + 1.0

    degp = _sc_scatter(ones_nd, src4d, dst4d, zeros)

    g = _tc_first(degp, x, W1)
    for b, w in ((b1, W2), (b2, W3), (b3, W4)):
        sp = _sc_scatter(g, src4d, dst4d, zeros)
        g = _tc_mid(degp, g, sp, b.reshape(1, D), w)
    sp = _sc_scatter(g, src4d, dst4d, zeros)
    out = _tc_final(degp, g, sp, b4.reshape(1, D))
    return out.reshape(N)


# speedup vs baseline: 14.4338x; 1.1659x over previous
import jax
import jax.numpy as jnp
from jax import lax
from jax.experimental import pallas as pl
from jax.experimental.pallas import tpu as pltpu
from jax.experimental.pallas import tpu_sc as plsc

N = 10000
E = 320000
D = 128

NC = 2
NS = 16
NW = NC * NS
EPW = E // NW
K = 125
NCHUNK = EPW // K
RSTRIPE = 624
RTAIL = N - NS * RSTRIPE


def _stripe_copy(src_at, dst_at, sid):
    pltpu.sync_copy(src_at(sid * RSTRIPE, RSTRIPE), dst_at(sid * RSTRIPE, RSTRIPE))

    @pl.when(sid == NS - 1)
    def _():
        pltpu.sync_copy(src_at(NS * RSTRIPE, RTAIL), dst_at(NS * RSTRIPE, RTAIL))


NB = 5
NROUND = NCHUNK // NB


def _sc_scatter_body(g_hbm, src4d_hbm, dst4d_hbm, zeros_hbm, out_hbm,
                     src_slab, dst_slab, acc_sh):
    cid = lax.axis_index("c")
    sid = lax.axis_index("s")
    wid = sid * NC + cid

    pltpu.sync_copy(src4d_hbm.at[wid], src_slab)
    pltpu.sync_copy(dst4d_hbm.at[wid], dst_slab)

    _stripe_copy(lambda o, n: zeros_hbm.at[pl.ds(o, n)],
                 lambda o, n: acc_sh.at[pl.ds(o, n)], sid)

    plsc.subcore_barrier()

    @plsc.parallel_loop(0, NCHUNK, unroll=5)
    def chunk(c):
        def inner(rows_v):
            pltpu.sync_copy(g_hbm.at[src_slab.at[c, 0]], rows_v)
            pltpu.sync_copy(rows_v, acc_sh.at[dst_slab.at[c, 0]], add=True)

        pl.run_scoped(inner, pltpu.VMEM((K, D), jnp.float32))

    plsc.subcore_barrier()

    _stripe_copy(lambda o, n: acc_sh.at[pl.ds(o, n)],
                 lambda o, n: out_hbm.at[cid, pl.ds(o, n)], sid)


_sc_scatter = pl.kernel(
    _sc_scatter_body,
    out_type=jax.ShapeDtypeStruct((NC, N, D), jnp.float32),
    mesh=plsc.VectorSubcoreMesh(core_axis_name="c", subcore_axis_name="s"),
    name="sc_edge_scatter",
    scratch_types=[
        pltpu.VMEM((NCHUNK, 1, K), jnp.int32),
        pltpu.VMEM((NCHUNK, 1, K), jnp.int32),
        pltpu.VMEM_SHARED((N, D), jnp.float32),
    ],
)


def _dinv(degp_ref):
    deg = degp_ref[0, :, 0:1] + degp_ref[1, :, 0:1] + 1.0
    return lax.rsqrt(deg)


def _tc_first_body(degp_ref, x_ref, w_ref, out_ref):
    dinv = _dinv(degp_ref)
    out_ref[...] = jnp.dot(dinv * x_ref[...], w_ref[...],
                           preferred_element_type=jnp.float32)


def _tc_mid_body(degp_ref, g_ref, sp_ref, b_ref, w_ref, out_ref):
    dinv = _dinv(degp_ref)
    h = dinv * (g_ref[...] + sp_ref[0] + sp_ref[1]) + b_ref[...]
    xn = jnp.where(h > 0, h, 0.1 * h)
    out_ref[...] = jnp.dot(dinv * xn, w_ref[...],
                           preferred_element_type=jnp.float32)


def _tc_final_body(degp_ref, g_ref, sp_ref, b_ref, out_ref):
    dinv = _dinv(degp_ref)
    h = dinv * (g_ref[...] + sp_ref[0] + sp_ref[1]) + b_ref[...]
    out_ref[...] = jnp.sum(h, axis=1, keepdims=True) * (1.0 / D)


_tc_first = pl.pallas_call(
    _tc_first_body, out_shape=jax.ShapeDtypeStruct((N, D), jnp.float32))
_tc_mid = pl.pallas_call(
    _tc_mid_body, out_shape=jax.ShapeDtypeStruct((N, D), jnp.float32))
_tc_final = pl.pallas_call(
    _tc_final_body, out_shape=jax.ShapeDtypeStruct((N, 1), jnp.float32))


def kernel(x, edge_index, W1, b1, W2, b2, W3, b3, W4, b4):
    src4d = edge_index[0].reshape(NW, NCHUNK, 1, K)
    dst4d = edge_index[1].reshape(NW, NCHUNK, 1, K)
    zeros = x * 0.0
    ones_nd = zeros + 1.0

    degp = _sc_scatter(ones_nd, src4d, dst4d, zeros)

    g = _tc_first(degp, x, W1)
    for b, w in ((b1, W2), (b2, W3), (b3, W4)):
        sp = _sc_scatter(g, src4d, dst4d, zeros)
        g = _tc_mid(degp, g, sp, b.reshape(1, D), w)
    sp = _sc_scatter(g, src4d, dst4d, zeros)
    out = _tc_final(degp, g, sp, b4.reshape(1, D))
    return out.reshape(N)
